# baseline (speedup 1.0000x reference)
import math

import jax
import jax.numpy as jnp
import numpy as np
from jax.experimental import pallas as pl
from jax.experimental.pallas import tpu as pltpu

B = 8
M = 256
EMBD = 128
K3D = 128
MAX_Z = 101
OFF = 128

_R = 8192
_A = (2 * 3.14159) ** 0.5
_INV_SQRT2 = 1.0 / math.sqrt(2.0)


_TWO_PI_PARTS = (6.28125, 0.0019353071693331003)
_INV_TWO_PI = float(np.float32(1.0 / (2.0 * np.pi)))
_SIN_COEFS = (2.1401396767539715e-06, -0.00019249443151001314,
              0.008307955164852027, -0.16662189927828033,
              0.9999778011834951)
_HALF_PI_SQ = float(np.float32((np.pi / 2.0) ** 2))


def _sincos_premul(phase, kf):
    r = phase
    for p in _TWO_PI_PARTS:
        r = r - kf * p
    s = r * r
    pol = _SIN_COEFS[0]
    for c in _SIN_COEFS[1:]:
        pol = pol * s + c
    sin_v = r * pol
    w = jnp.maximum(1.0 - sin_v * sin_v, 1e-30)
    cmag = w * jax.lax.rsqrt(w)
    cos_v = jnp.where(s < _HALF_PI_SQ, cmag, -cmag)
    return sin_v, cos_v


def _acos(z):
    return jnp.arctan2(jnp.sqrt(jnp.maximum(1.0 - z * z, 0.0)), z)


def _geom_kernel(pos_col_ref, pos_row_ref, d_ref, az_ref, pol_ref):
    pc = pos_col_ref[0]
    pr = pos_row_ref[0]
    dx = pr[0:1, :] - pc[:, 0:1]
    dy = pr[1:2, :] - pc[:, 1:2]
    dz = pr[2:3, :] - pc[:, 2:3]
    s = dx * dx + dy * dy + dz * dz
    d_ref[0] = jnp.sqrt(s + 1e-12)
    az_ref[0] = jnp.arctan2(dy, dx)
    ndz = dz / (jnp.sqrt(s) + 1e-5)
    pol_ref[0] = _acos(jnp.clip(ndz, -1.0, 1.0))


def _pair_kernel(d_ref, az_ref, pol_ref, mb_ref, means_ref, stds_ref,
                 l1w_ref, l1b_ref, w2_ref, b2_ref, fr_ref, out_ref):
    d = d_ref[...]
    az = az_ref[...]
    po = pol_ref[...]
    mul = mb_ref[0, 0]
    bias = mb_ref[0, 1]
    std = jnp.abs(stds_ref[...]) + 0.01
    inv_std = 1.0 / std
    _KE = 0.8493218002880191
    ac = (mul * _KE) * inv_std
    cc = (bias - means_ref[...]) * inv_std * _KE
    c2 = -jnp.log2(_A * std)
    arg = d * ac + cc
    gk = jnp.exp2(c2 - arg * arg)
    hid2 = jnp.dot(gk.astype(jnp.bfloat16), l1w_ref[...],
                   preferred_element_type=jnp.float32)
    hid2 = hid2 + l1b_ref[...]
    h2 = _INV_SQRT2 * hid2
    hid = h2 * jax.lax.erf(hid2) + h2
    fr = fr_ref[...]
    azpo = jnp.concatenate(
        [jnp.broadcast_to(az, (az.shape[0], 64)),
         jnp.broadcast_to(po, (po.shape[0], 64))], axis=1)
    ph = azpo * fr
    kf = jnp.floor(ph * _INV_TWO_PI + 0.5)
    sin_c, cos_c = _sincos_premul(ph, kf)
    feats = jnp.concatenate(
        [hid.astype(jnp.bfloat16),
         sin_c.astype(jnp.bfloat16), cos_c.astype(jnp.bfloat16)],
        axis=1)
    e = jnp.dot(feats, w2_ref[...], preferred_element_type=jnp.float32)
    out_ref[...] = e + b2_ref[...]


def _h_kernel(azc_ref, table_ref, elec101_ref, mult_ref, chg_ref,
              multtab_ref, chgtab_ref, out_ref):
    azc = azc_ref[...]
    lane = jax.lax.broadcasted_iota(jnp.int32, (1, 128), 1)
    onehot = (azc == lane).astype(jnp.float32)
    h = jnp.dot(onehot, table_ref[...], preferred_element_type=jnp.float32)
    moh = (mult_ref[...] == lane).astype(jnp.float32)
    coh = ((chg_ref[...] + OFF // 2) == lane).astype(jnp.float32)
    g = jnp.dot(moh, multtab_ref[...], preferred_element_type=jnp.float32)
    g = g + jnp.dot(coh, chgtab_ref[...], preferred_element_type=jnp.float32)
    r = jax.lax.broadcasted_iota(jnp.int32, (B * M, 1), 0)
    is_cls = (r % M == 0).astype(jnp.float32)
    boh = ((r // M) == jax.lax.broadcasted_iota(jnp.int32, (1, B), 1))
    gb = jnp.dot(boh.astype(jnp.float32), g,
                 preferred_element_type=jnp.float32)
    out_ref[...] = h + is_cls * (gb - elec101_ref[...])


def kernel(positions, atomic_numbers, mask, multiplicity, charge, emb_table,
           electron_config, cfg_W, cfg_b, mult_table, charge_table, means,
           stds, mul_w, bias_w, l1_W, l1_b, l2_W, l2_b, freqs_az, freqs_po,
           proj_W, proj_b):
    f32 = jnp.float32
    pos = jnp.concatenate([jnp.zeros_like(positions[:, :1]), positions], 1)
    az_full = jnp.concatenate(
        [jnp.full_like(atomic_numbers[:, :1], MAX_Z), atomic_numbers], 1)
    msk = jnp.concatenate([jnp.ones_like(mask[:, :1]), mask], 1)

    pos_row = jnp.transpose(pos, (0, 2, 1))
    d, azm, pol = pl.pallas_call(
        _geom_kernel,
        grid=(B,),
        in_specs=[
            pl.BlockSpec((1, M, 3), lambda b: (b, 0, 0)),
            pl.BlockSpec((1, 3, M), lambda b: (b, 0, 0)),
        ],
        out_specs=[pl.BlockSpec((1, M, M), lambda b: (b, 0, 0))] * 3,
        out_shape=[jax.ShapeDtypeStruct((B, M, M), f32)] * 3,
    )(pos, pos_row)

    nrows = B * M * M
    grid = nrows // _R
    d_c = d.reshape(nrows, 1)
    az_c = azm.reshape(nrows, 1)
    pol_c = pol.reshape(nrows, 1)
    mb = jnp.stack([mul_w[0, 0], bias_w[0, 0]]).reshape(1, 2)
    col = pl.BlockSpec((_R, 1), lambda g: (g, 0))
    full = lambda shape: pl.BlockSpec(shape, lambda g: (0,) * len(shape))
    bf16 = jnp.bfloat16
    pt = proj_W.T
    w2 = jnp.concatenate(
        [l2_W.T, pt[0:64], pt[128:192], pt[64:128], pt[192:256]],
        axis=0).astype(bf16)
    b2 = (l2_b + proj_b).reshape(1, EMBD)
    e_flat = pl.pallas_call(
        _pair_kernel,
        grid=(grid,),
        in_specs=[
            col, col, col,
            full((1, 2)),
            full((1, K3D)), full((1, K3D)),
            full((K3D, K3D)), full((1, K3D)),
            full((K3D + 256, EMBD)), full((1, EMBD)),
            full((1, 128)),
        ],
        out_specs=pl.BlockSpec((_R, EMBD), lambda g: (g, 0)),
        out_shape=jax.ShapeDtypeStruct((nrows, EMBD), f32),
        compiler_params=pltpu.CompilerParams(
            dimension_semantics=("parallel",)),
    )(d_c, az_c, pol_c, mb, means.reshape(1, K3D), stds.reshape(1, K3D),
      (l1_W.T * _INV_SQRT2).astype(bf16),
      (l1_b * _INV_SQRT2).reshape(1, K3D), w2, b2,
      jnp.concatenate([freqs_az, freqs_po]).reshape(1, 128))
    e = e_flat.reshape(B, M, M, EMBD)

    pad = 128 - (MAX_Z + 1)
    emb_pad = jnp.pad(emb_table, ((0, pad), (0, 0)))
    ec_pad = jnp.pad(electron_config, ((0, pad), (0, 0)))
    azc = az_full.reshape(B * M, 1)
    h_flat = pl.pallas_call(
        _h_table_call,
        grid=(1,),
        in_specs=[
            pl.BlockSpec((B * M, 1), lambda g: (0, 0)),
            full((128, EMBD)), full((128, 20)), full((20, EMBD)),
            full((1, EMBD)), full((B, 1)), full((B, 1)),
            full((OFF, EMBD)), full((OFF, EMBD)),
        ],
        out_specs=pl.BlockSpec((B * M, EMBD), lambda g: (0, 0)),
        out_shape=jax.ShapeDtypeStruct((B * M, EMBD), f32),
    )(azc, emb_pad, ec_pad, cfg_W.T, cfg_b.reshape(1, EMBD), multiplicity,
      charge, mult_table, charge_table)
    h = h_flat.reshape(B, M, EMBD)
    return (h, e, msk)


def _h_table_call(azc_ref, emb_ref, ec_ref, cfgwt_ref, cfgb_ref, mult_ref,
                  chg_ref, multtab_ref, chgtab_ref, out_ref):
    elec = jnp.dot(ec_ref[...], cfgwt_ref[...],
                   preferred_element_type=jnp.float32) + cfgb_ref[...]
    table = emb_ref[...] + elec
    _h_kernel(azc_ref, _Const(table), _Const(elec[MAX_Z:MAX_Z + 1, :]),
              mult_ref, chg_ref, multtab_ref, chgtab_ref, out_ref)


class _Const:
    def __init__(self, v):
        self._v = v

    def __getitem__(self, idx):
        return self._v


# speedup vs baseline: 1.9895x; 1.9895x over previous
import math

import jax
import jax.numpy as jnp
import numpy as np
from jax.experimental import pallas as pl
from jax.experimental.pallas import tpu as pltpu

B = 8
M = 256
EMBD = 128
K3D = 128
MAX_Z = 101
OFF = 128

_R = 4096
_A = (2 * 3.14159) ** 0.5
_INV_SQRT2 = 1.0 / math.sqrt(2.0)


_TWO_PI_PARTS = (6.28125, 0.0019353071693331003)
_INV_TWO_PI = float(np.float32(1.0 / (2.0 * np.pi)))
_SIN_COEFS = (2.1401396767539715e-06, -0.00019249443151001314,
              0.008307955164852027, -0.16662189927828033,
              0.9999778011834951)
_HALF_PI_SQ = float(np.float32((np.pi / 2.0) ** 2))


def _sincos_premul(phase, kf):
    r = phase
    for p in _TWO_PI_PARTS:
        r = r - kf * p
    s = r * r
    pol = _SIN_COEFS[0]
    for c in _SIN_COEFS[1:]:
        pol = pol * s + c
    sin_v = r * pol
    w = jnp.maximum(1.0 - sin_v * sin_v, 1e-30)
    cmag = w * jax.lax.rsqrt(w)
    cos_v = jnp.where(s < _HALF_PI_SQ, cmag, -cmag)
    return sin_v, cos_v


def _acos(z):
    return jnp.arctan2(jnp.sqrt(jnp.maximum(1.0 - z * z, 0.0)), z)


def _geom_kernel(pos_col_ref, pos_row_ref, d_ref, az_ref, pol_ref):
    pc = pos_col_ref[0]
    pr = pos_row_ref[0]
    dx = pr[0:1, :] - pc[:, 0:1]
    dy = pr[1:2, :] - pc[:, 1:2]
    dz = pr[2:3, :] - pc[:, 2:3]
    s = dx * dx + dy * dy + dz * dz
    d_ref[0] = jnp.sqrt(s + 1e-12)
    az_ref[0] = jnp.arctan2(dy, dx)
    ndz = dz / (jnp.sqrt(s) + 1e-5)
    pol_ref[0] = _acos(jnp.clip(ndz, -1.0, 1.0))


def _pair_kernel(geo_ref, mb_ref, means_ref, stds_ref,
                 l1w_ref, l1b_ref, w2_ref, b2_ref, fr_ref, out_ref):
    geo = geo_ref[...]
    d = geo[:, 0:1]
    az = geo[:, 1:2]
    po = geo[:, 2:3]
    mul = mb_ref[0, 0]
    bias = mb_ref[0, 1]
    std = jnp.abs(stds_ref[...]) + 0.01
    inv_std = 1.0 / std
    _KE = 0.8493218002880191
    ac = (mul * _KE) * inv_std
    cc = (bias - means_ref[...]) * inv_std * _KE
    c2 = -jnp.log2(_A * std)
    arg = d * ac + cc
    gk = jnp.exp2(c2 - arg * arg)
    hid2 = jnp.dot(gk.astype(jnp.bfloat16), l1w_ref[...],
                   preferred_element_type=jnp.float32)
    hid2 = hid2 + l1b_ref[...]
    h2 = _INV_SQRT2 * hid2
    hid = h2 * jax.lax.erf(hid2) + h2
    fr = fr_ref[...]
    azpo = jnp.concatenate(
        [jnp.broadcast_to(az, (az.shape[0], 64)),
         jnp.broadcast_to(po, (po.shape[0], 64))], axis=1)
    ph = azpo * fr
    kf = jnp.floor(ph * _INV_TWO_PI + 0.5)
    sin_c, cos_c = _sincos_premul(ph, kf)
    feats = jnp.concatenate(
        [hid.astype(jnp.bfloat16),
         sin_c.astype(jnp.bfloat16), cos_c.astype(jnp.bfloat16)],
        axis=1)
    e = jnp.dot(feats, w2_ref[...], preferred_element_type=jnp.float32)
    out_ref[...] = e + b2_ref[...]


def _h_kernel(azc_ref, table_ref, elec101_ref, mult_ref, chg_ref,
              multtab_ref, chgtab_ref, out_ref):
    azc = azc_ref[...]
    lane = jax.lax.broadcasted_iota(jnp.int32, (1, 128), 1)
    onehot = (azc == lane).astype(jnp.float32)
    h = jnp.dot(onehot, table_ref[...], preferred_element_type=jnp.float32)
    moh = (mult_ref[...] == lane).astype(jnp.float32)
    coh = ((chg_ref[...] + OFF // 2) == lane).astype(jnp.float32)
    g = jnp.dot(moh, multtab_ref[...], preferred_element_type=jnp.float32)
    g = g + jnp.dot(coh, chgtab_ref[...], preferred_element_type=jnp.float32)
    r = jax.lax.broadcasted_iota(jnp.int32, (B * M, 1), 0)
    is_cls = (r % M == 0).astype(jnp.float32)
    boh = ((r // M) == jax.lax.broadcasted_iota(jnp.int32, (1, B), 1))
    gb = jnp.dot(boh.astype(jnp.float32), g,
                 preferred_element_type=jnp.float32)
    out_ref[...] = h + is_cls * (gb - elec101_ref[...])


def kernel(positions, atomic_numbers, mask, multiplicity, charge, emb_table,
           electron_config, cfg_W, cfg_b, mult_table, charge_table, means,
           stds, mul_w, bias_w, l1_W, l1_b, l2_W, l2_b, freqs_az, freqs_po,
           proj_W, proj_b):
    f32 = jnp.float32
    pos = jnp.concatenate([jnp.zeros_like(positions[:, :1]), positions], 1)
    az_full = jnp.concatenate(
        [jnp.full_like(atomic_numbers[:, :1], MAX_Z), atomic_numbers], 1)
    msk = jnp.concatenate([jnp.ones_like(mask[:, :1]), mask], 1)

    pos_row = jnp.transpose(pos, (0, 2, 1))
    d, azm, pol = pl.pallas_call(
        _geom_kernel,
        grid=(B,),
        in_specs=[
            pl.BlockSpec((1, M, 3), lambda b: (b, 0, 0)),
            pl.BlockSpec((1, 3, M), lambda b: (b, 0, 0)),
        ],
        out_specs=[pl.BlockSpec((1, M, M), lambda b: (b, 0, 0))] * 3,
        out_shape=[jax.ShapeDtypeStruct((B, M, M), f32)] * 3,
    )(pos, pos_row)

    nrows = B * M * M
    grid = nrows // _R
    geo = jnp.stack([d, azm, pol], axis=-1).reshape(nrows, 3)
    mb = jnp.stack([mul_w[0, 0], bias_w[0, 0]]).reshape(1, 2)
    col = pl.BlockSpec((_R, 3), lambda g: (g, 0))
    full = lambda shape: pl.BlockSpec(shape, lambda g: (0,) * len(shape))
    bf16 = jnp.bfloat16
    pt = proj_W.T
    w2 = jnp.concatenate(
        [l2_W.T, pt[0:64], pt[128:192], pt[64:128], pt[192:256]],
        axis=0).astype(bf16)
    b2 = (l2_b + proj_b).reshape(1, EMBD)
    e_flat = pl.pallas_call(
        _pair_kernel,
        grid=(grid,),
        in_specs=[
            col,
            full((1, 2)),
            full((1, K3D)), full((1, K3D)),
            full((K3D, K3D)), full((1, K3D)),
            full((K3D + 256, EMBD)), full((1, EMBD)),
            full((1, 128)),
        ],
        out_specs=pl.BlockSpec((_R, EMBD), lambda g: (g, 0)),
        out_shape=jax.ShapeDtypeStruct((nrows, EMBD), f32),
        compiler_params=pltpu.CompilerParams(
            dimension_semantics=("parallel",)),
    )(geo, mb, means.reshape(1, K3D), stds.reshape(1, K3D),
      (l1_W.T * _INV_SQRT2).astype(bf16),
      (l1_b * _INV_SQRT2).reshape(1, K3D), w2, b2,
      jnp.concatenate([freqs_az, freqs_po]).reshape(1, 128))
    e = e_flat.reshape(B, M, M, EMBD)

    pad = 128 - (MAX_Z + 1)
    emb_pad = jnp.pad(emb_table, ((0, pad), (0, 0)))
    ec_pad = jnp.pad(electron_config, ((0, pad), (0, 0)))
    azc = az_full.reshape(B * M, 1)
    h_flat = pl.pallas_call(
        _h_table_call,
        grid=(1,),
        in_specs=[
            pl.BlockSpec((B * M, 1), lambda g: (0, 0)),
            full((128, EMBD)), full((128, 20)), full((20, EMBD)),
            full((1, EMBD)), full((B, 1)), full((B, 1)),
            full((OFF, EMBD)), full((OFF, EMBD)),
        ],
        out_specs=pl.BlockSpec((B * M, EMBD), lambda g: (0, 0)),
        out_shape=jax.ShapeDtypeStruct((B * M, EMBD), f32),
    )(azc, emb_pad, ec_pad, cfg_W.T, cfg_b.reshape(1, EMBD), multiplicity,
      charge, mult_table, charge_table)
    h = h_flat.reshape(B, M, EMBD)
    return (h, e, msk)


def _h_table_call(azc_ref, emb_ref, ec_ref, cfgwt_ref, cfgb_ref, mult_ref,
                  chg_ref, multtab_ref, chgtab_ref, out_ref):
    elec = jnp.dot(ec_ref[...], cfgwt_ref[...],
                   preferred_element_type=jnp.float32) + cfgb_ref[...]
    table = emb_ref[...] + elec
    _h_kernel(azc_ref, _Const(table), _Const(elec[MAX_Z:MAX_Z + 1, :]),
              mult_ref, chg_ref, multtab_ref, chgtab_ref, out_ref)


class _Const:
    def __init__(self, v):
        self._v = v

    def __getitem__(self, idx):
        return self._v
